# baseline (speedup 1.0000x reference)
import functools

import jax
import jax.numpy as jnp
from jax import lax
from jax.experimental import pallas as pl
from jax.experimental.pallas import tpu as pltpu
from jax.experimental.pallas import tpu_sc as plsc

VOCAB = 1000000
DIM = 64
BATCH = 16384
NEG = 20

NC = 2
NS = 16
NW = NC * NS
BPW = BATCH // NW
C = 32
NCHUNK = BPW // C
KROWS = C * NEG
KG = KROWS // 128


def _body(u_hbm, v_hbm, cen_hbm, ctx_hbm, out_hbm,
          cidx, uidx, v_rows, u_rows, out_v, sem_v, sem_u):
    wid = lax.axis_index("s") * NC + lax.axis_index("c")
    lane = lax.iota(jnp.int32, 16)

    def chunk(g, _):
        pltpu.sync_copy(cen_hbm.at[wid, g], cidx)
        pltpu.sync_copy(ctx_hbm.at[wid, g], uidx)
        pltpu.async_copy(v_hbm.at[cidx], v_rows, sem_v)
        for j in range(KG):
            pltpu.async_copy(u_hbm.at[uidx.at[j]],
                             u_rows.at[pl.ds(j * 128, 128)], sem_u)
        pltpu.make_async_copy(v_hbm.at[pl.ds(0, C)], v_rows, sem_v).wait()
        pltpu.make_async_copy(u_hbm.at[pl.ds(0, KROWS)], u_rows, sem_u).wait()

        for grp in range(C // 16):
            bvec = lane + grp * 16
            rbase = bvec * NEG

            def dstep(d, acc):
                dvec = jnp.full((16,), d, jnp.int32)
                vv = plsc.load_gather(v_rows, [bvec, dvec])
                return tuple(
                    acc[l] + plsc.load_gather(u_rows, [rbase + l, dvec]) * vv
                    for l in range(NEG))

            acc = lax.fori_loop(
                0, DIM, dstep,
                tuple(jnp.zeros((16,), jnp.float32) for _ in range(NEG)))
            for l in range(NEG):
                plsc.store_scatter(out_v, [rbase + l], acc[l])

        def sig(i, _):
            x = out_v[pl.ds(i * 16, 16)]
            out_v[pl.ds(i * 16, 16)] = 1.0 / (1.0 + jnp.exp(-x))
            return 0

        lax.fori_loop(0, KROWS // 16, sig, 0)
        pltpu.sync_copy(out_v, out_hbm.at[wid, g])
        return 0

    lax.fori_loop(0, NCHUNK, chunk, 0)


@jax.jit
def _run(u_embeds, v_embeds, cen, ctx):
    mesh = plsc.VectorSubcoreMesh(core_axis_name="c", subcore_axis_name="s")
    f = pl.kernel(
        _body,
        out_type=jax.ShapeDtypeStruct((NW, NCHUNK, KROWS), jnp.float32),
        mesh=mesh,
        compiler_params=pltpu.CompilerParams(
            needs_layout_passes=False, use_tc_tiling_on_sc=False),
        scratch_types=[
            pltpu.VMEM((C,), jnp.int32),
            pltpu.VMEM((KG, 128), jnp.int32),
            pltpu.VMEM((C, DIM), jnp.float32),
            pltpu.VMEM((KROWS, DIM), jnp.float32),
            pltpu.VMEM((KROWS,), jnp.float32),
            pltpu.SemaphoreType.DMA,
            pltpu.SemaphoreType.DMA,
        ],
    )
    return f(u_embeds, v_embeds, cen, ctx)


def kernel(u_embeds, v_embeds, centers, context_and_negatives):
    cen = jnp.asarray(centers, jnp.int32).reshape(NW, NCHUNK, C)
    ctx = jnp.asarray(context_and_negatives, jnp.int32).reshape(
        NW, NCHUNK, KG, 128)
    out = _run(u_embeds, v_embeds, cen, ctx)
    return out.reshape(BATCH, NEG)
